# Initial kernel scaffold; baseline (speedup 1.0000x reference)
import jax
import jax.numpy as jnp
from jax.experimental import pallas as pl


def kernel(features, anchors, w1, b1, w_obj, b_obj, w_del, b_del):
    raise NotImplementedError("write your pallas kernel here")


# speedup vs baseline: 7.0165x; 7.0165x over previous
import jax
import jax.numpy as jnp
import numpy as np
from jax import lax
from jax.experimental import pallas as pl
from jax.experimental.pallas import tpu as pltpu

H = 64
W = 64
C = 256
HW = H * W
N = HW * 3
CAP = 2048
PRE_K = 2000
POST_K = 1000
ROWB = 1024
NMS_THRESH = 0.7
SCALE_CLAMP = float(np.log(1000.0 / 16.0))
NEG_BIG = -1e30

_TREES = [
    "((((((((0+1)+2)+3)+4)+5)+6)+7)+8)",
    "(((((((0+(1+2))+3)+4)+5)+6)+7)+8)",
    "(((((((0+1)+(2+3))+4)+5)+6)+7)+8)",
    "(((((((0+1)+2)+(3+4))+5)+6)+7)+8)",
    "(((((((0+1)+2)+3)+(4+5))+6)+7)+8)",
    "(((((((0+1)+2)+3)+4)+(5+6))+7)+8)",
    "(((((((0+1)+2)+3)+4)+5)+(6+7))+8)",
    "(((((((0+1)+2)+3)+4)+5)+6)+(7+8))",
    "((((((0+1)+2)+3)+((4+5)+6))+7)+8)",
    "(((((0+((1+2)+3))+4)+5)+(6+7))+8)",
    "(((((0+1)+((2+3)+4))+5)+6)+(7+8))",
    "(((((0+1)+2)+3)+4)+5)",
    "(((3+((4+5)+6))+7)+8)",
]
_CLASS_ROWS = (
    "ccccccccc", "888888888", "888844444", "444444444", "444444444",
    "444444444", "444444444", "444444444", "444077777", "777777777",
    "777777777", "777777777", "777777777", "777777777", "777666666",
    "666666666", "666666666", "666666666", "666666666", "666666666",
    "665555555", "555555555", "555555555", "555555555", "555555555",
    "555555555", "544444444", "444444444", "444444444", "444444444",
    "444444444", "444444444", "000000000", "000000000", "000000000",
    "000000000", "000000000", "000000000", "000000000", "000000000",
    "000033333", "333333333", "333333333", "333333333", "333333333",
    "333333333", "333aaaaaa", "aaaaaaaaa", "aaaaaaa22", "222222222",
    "222222222", "222222222", "229999999", "999999999", "999999111",
    "111111111", "111111111", "111111111", "155555555", "555555555",
    "555550000", "000000000", "000000000", "bbbbbbbbb",
)


def _parse_tree(s, pos=0):
    if s[pos] != "(":
        return int(s[pos]), pos + 1
    left, pos = _parse_tree(s, pos + 1)
    assert s[pos] == "+"
    right, pos = _parse_tree(s, pos + 1)
    assert s[pos] == ")"
    return (left, right), pos + 1


_TREE_AST = [_parse_tree(s)[0] for s in _TREES]
_cm = np.array([[int(c, 16) for c in row] for row in _CLASS_ROWS])
_p = np.arange(HW)
_CLS_NP = _cm[_p // W, (_p % W) % 9].astype(np.int32).reshape(HW, 1)


def _eval_tree(t, parts):
    if isinstance(t, int):
        return parts[t]
    return _eval_tree(t[0], parts) + _eval_tree(t[1], parts)


def _shifted(xf, k):
    kh, kw = k // 3, k % 3
    dy, dx = kh - 1, kw - 1
    off = dy * W + dx
    if off > 0:
        sh = jnp.concatenate([xf[off:], jnp.zeros((off, C), xf.dtype)], axis=0)
    elif off < 0:
        sh = jnp.concatenate([jnp.zeros((-off, C), xf.dtype), xf[:off]], axis=0)
    else:
        sh = xf
    if dx != 0:
        p = lax.broadcasted_iota(jnp.int32, (HW, 1), 0)
        col = p % W + dx
        sh = jnp.where((col >= 0) & (col < W), sh, 0.0)
    return sh


def _dotf(a, b):
    return lax.dot_general(a, b, (((1,), (0,)), ((), ())),
                           preferred_element_type=jnp.float32)


CHUNK = 1024


def conv_head_body(x_ref, w_ref, b1_ref, w2_ref, b2_ref, cls_ref, o_ref):
    g = pl.program_id(0)
    xv = x_ref[pl.ds(g * CHUNK, CHUNK + 256), :]
    prow = lax.broadcasted_iota(jnp.int32, (CHUNK, 1), 0) + g * CHUNK
    parts = []
    for k in range(9):
        kh, kw = k // 3, k % 3
        dy, dx = kh - 1, kw - 1
        off = dy * W + dx
        sh = xv[128 + off:128 + off + CHUNK, :]
        if dx != 0:
            col = prow % W + dx
            sh = jnp.where((col >= 0) & (col < W), sh, 0.0)
        wk = w_ref[k * C:(k + 1) * C, :].astype(jnp.bfloat16)
        parts.append(_dotf(sh.astype(jnp.bfloat16), wk))
    cls = cls_ref[...]
    acc = jnp.zeros((CHUNK, C), jnp.float32)
    for cid, ast in enumerate(_TREE_AST):
        acc = jnp.where(cls == cid, _eval_tree(ast, parts), acc)
    t = jnp.maximum(acc + b1_ref[...], 0.0)
    o_ref[...] = _dotf(t.astype(jnp.bfloat16),
                       w2_ref[...].astype(jnp.bfloat16)) + b2_ref[...]


def decode_body(dl_ref, an_ref, o_ref):
    deltas = dl_ref[...]
    a = an_ref[...]
    widths = a[:, 2] - a[:, 0]
    heights = a[:, 3] - a[:, 1]
    ctr_x = a[:, 0] + 0.5 * widths
    ctr_y = a[:, 1] + 0.5 * heights
    dx, dy = deltas[:, 0], deltas[:, 1]
    dw = jnp.minimum(deltas[:, 2], SCALE_CLAMP)
    dh = jnp.minimum(deltas[:, 3], SCALE_CLAMP)
    pcx = dx * widths + ctr_x
    pcy = dy * heights + ctr_y
    pw = jnp.exp(dw) * widths
    ph = jnp.exp(dh) * heights
    x1 = jnp.clip(pcx - 0.5 * pw, 0.0, 512.0)
    y1 = jnp.clip(pcy - 0.5 * ph, 0.0, 512.0)
    x2 = jnp.clip(pcx + 0.5 * pw, 0.0, 512.0)
    y2 = jnp.clip(pcy + 0.5 * ph, 0.0, 512.0)
    o_ref[...] = jnp.stack([x1, y1, x2, y2], axis=1)


def _monokey(s):
    b = lax.bitcast_convert_type(s, jnp.int32)
    return jnp.where(b < 0, b ^ jnp.int32(0x7FFFFFFF), b)


def rank_all_body(col_ref, row_ref, o_ref):
    g = pl.program_id(0)
    c = pl.program_id(1)
    ka = _monokey(col_ref[...])
    kb = _monokey(row_ref[...])
    ia = lax.broadcasted_iota(jnp.int32, (ROWB, 1), 0) + g * ROWB
    ib = lax.broadcasted_iota(jnp.int32, (1, CAP), 1) + c * CAP
    gt = (kb > ka) | ((kb == ka) & (ib < ia))
    part = jnp.sum(gt.astype(jnp.int32), axis=1, keepdims=True)

    @pl.when(c == 0)
    def _():
        o_ref[...] = part

    @pl.when(c != 0)
    def _():
        o_ref[...] = o_ref[...] + part


def _split3(v):
    a = v.astype(jnp.bfloat16)
    r1 = v - a.astype(jnp.float32)
    b = r1.astype(jnp.bfloat16)
    cc = (r1 - b.astype(jnp.float32)).astype(jnp.bfloat16)
    return a, b, cc


def sel_body(rank_ref, vals_ref, o_ref):
    c = pl.program_id(0)
    rk = rank_ref[...]
    r_iota = lax.broadcasted_iota(jnp.int32, (CAP, 1), 0)
    oh = (rk == r_iota).astype(jnp.bfloat16)
    va, vb, vc = _split3(vals_ref[...])
    part = (_dotf(oh, va) + _dotf(oh, vb)) + _dotf(oh, vc)

    @pl.when(c == 0)
    def _():
        o_ref[...] = part

    @pl.when(c != 0)
    def _():
        o_ref[...] = o_ref[...] + part


def nms_body(bc_ref, sc_ref, o_ref, iou_ref):
    colmask = lax.broadcasted_iota(jnp.int32, (CAP, 1), 0) < PRE_K
    boxes = jnp.where(colmask, bc_ref[...], 0.0)
    area = jnp.maximum(boxes[:, 2] - boxes[:, 0], 0.0) * \
        jnp.maximum(boxes[:, 3] - boxes[:, 1], 0.0)
    RB = 256
    for rb in range(0, CAP, RB):
        bb = boxes[rb:rb + RB]
        ltx = jnp.maximum(bb[:, None, 0], boxes[None, :, 0])
        lty = jnp.maximum(bb[:, None, 1], boxes[None, :, 1])
        rbx = jnp.minimum(bb[:, None, 2], boxes[None, :, 2])
        rby = jnp.minimum(bb[:, None, 3], boxes[None, :, 3])
        wh_x = jnp.maximum(rbx - ltx, 0.0)
        wh_y = jnp.maximum(rby - lty, 0.0)
        inter = wh_x * wh_y
        union = area[rb:rb + RB, None] + area[None, :] - inter
        iou_ref[rb:rb + RB, :] = inter / jnp.maximum(union, 1e-9)

    col_i = lax.broadcasted_iota(jnp.int32, (1, CAP), 1)

    def body(g8, alive):
        grp = iou_ref[pl.ds(g8 * 8, 8), :]
        for j in range(8):
            i = g8 * 8 + j
            row = grp[j:j + 1, :]
            ki = jnp.sum(jnp.where((col_i == i) & (alive > 0), 1, 0)) > 0
            supp = (row > NMS_THRESH) & (col_i > i) & ki
            alive = jnp.where(supp, 0, alive)
        return alive

    alive = lax.fori_loop(0, PRE_K // 8, body, jnp.ones((1, CAP), jnp.int32))
    rowmask = col_i < PRE_K
    o_ref[...] = jnp.where(
        rowmask, jnp.where(alive > 0, sc_ref[...], -1e4), NEG_BIG)


def rank2_body(col_ref, row_ref, o_ref):
    a = col_ref[...]
    b = row_ref[...]
    icol = lax.broadcasted_iota(jnp.int32, (CAP, 1), 0)
    irow = lax.broadcasted_iota(jnp.int32, (1, CAP), 1)
    gt = (b > a) | ((b == a) & (irow < icol))
    o_ref[...] = jnp.sum(gt.astype(jnp.int32), axis=1, keepdims=True)


def fin_body(rank_ref, vals_ref, o_ref):
    rk = rank_ref[...]
    r_iota = lax.broadcasted_iota(jnp.int32, (POST_K + 24, 1), 0)
    oh = (rk == r_iota).astype(jnp.bfloat16)
    va, vb, vc = _split3(vals_ref[...])
    o_ref[...] = (_dotf(oh, va) + _dotf(oh, vb)) + _dotf(oh, vc)


def kernel(features, anchors, w1, b1, w_obj, b_obj, w_del, b_del):
    f32 = jnp.float32
    xb = jnp.transpose(features[0], (1, 2, 0)).reshape(HW, C)
    wt = jnp.transpose(w1, (2, 3, 1, 0)).reshape(9 * C, C)
    w2 = jnp.zeros((C, 128), f32)
    w2 = w2.at[:, 0:3].set(w_obj[:, :, 0, 0].T)
    w2 = w2.at[:, 3:15].set(w_del[:, :, 0, 0].T)
    b2 = jnp.zeros((1, 128), f32)
    b2 = b2.at[0, 0:3].set(b_obj)
    b2 = b2.at[0, 3:15].set(b_del)
    cls = jnp.asarray(_CLS_NP)

    xpad = jnp.concatenate(
        [jnp.zeros((128, C), f32), xb, jnp.zeros((128, C), f32)], axis=0)
    out128 = pl.pallas_call(
        conv_head_body,
        grid=(HW // CHUNK,),
        in_specs=[pl.BlockSpec((HW + 256, C), lambda g: (0, 0)),
                  pl.BlockSpec((9 * C, C), lambda g: (0, 0)),
                  pl.BlockSpec((1, C), lambda g: (0, 0)),
                  pl.BlockSpec((C, 128), lambda g: (0, 0)),
                  pl.BlockSpec((1, 128), lambda g: (0, 0)),
                  pl.BlockSpec((CHUNK, 1), lambda g: (g, 0))],
        out_specs=pl.BlockSpec((CHUNK, 128), lambda g: (g, 0)),
        out_shape=jax.ShapeDtypeStruct((HW, 128), f32),
    )(xpad, wt, b1.reshape(1, C), w2, b2, cls)

    logits = out128[:, 0:3].reshape(N)
    deltas4 = out128[:, 3:15].reshape(N, 4)

    boxes = pl.pallas_call(
        decode_body,
        out_shape=jax.ShapeDtypeStruct((N, 4), f32),
    )(deltas4, anchors)

    rank = pl.pallas_call(
        rank_all_body,
        grid=(N // ROWB, N // CAP),
        in_specs=[pl.BlockSpec((ROWB, 1), lambda g, c: (g, 0)),
                  pl.BlockSpec((1, CAP), lambda g, c: (0, c))],
        out_specs=pl.BlockSpec((ROWB, 1), lambda g, c: (g, 0)),
        out_shape=jax.ShapeDtypeStruct((N, 1), jnp.int32),
    )(logits.reshape(N, 1), logits.reshape(1, N))

    vals = jnp.concatenate(
        [logits.reshape(N, 1), boxes, jnp.zeros((N, 3), f32)], axis=1)
    sel = pl.pallas_call(
        sel_body,
        grid=(N // CAP,),
        in_specs=[pl.BlockSpec((1, CAP), lambda c: (0, c)),
                  pl.BlockSpec((CAP, 8), lambda c: (c, 0))],
        out_specs=pl.BlockSpec((CAP, 8), lambda c: (0, 0)),
        out_shape=jax.ShapeDtypeStruct((CAP, 8), f32),
    )(rank.reshape(1, N), vals)

    ssc = sel[:, 0]
    bc = sel[:, 1:5]
    scm = pl.pallas_call(
        nms_body,
        out_shape=jax.ShapeDtypeStruct((1, CAP), f32),
        scratch_shapes=[pltpu.VMEM((CAP, CAP), f32)],
    )(bc, ssc.reshape(1, CAP))

    rank2 = pl.pallas_call(
        rank2_body,
        out_shape=jax.ShapeDtypeStruct((CAP, 1), jnp.int32),
    )(scm.reshape(CAP, 1), scm)

    vals2 = jnp.concatenate(
        [scm.reshape(CAP, 1), bc, jnp.zeros((CAP, 3), f32)], axis=1)
    fin = pl.pallas_call(
        fin_body,
        out_shape=jax.ShapeDtypeStruct((POST_K + 24, 8), f32),
    )(rank2.reshape(1, CAP), vals2)

    return jnp.concatenate(
        [fin[:POST_K, 1:5], fin[:POST_K, 0:1]], axis=1)


# speedup vs baseline: 7.0853x; 1.0098x over previous
import jax
import jax.numpy as jnp
import numpy as np
from jax import lax
from jax.experimental import pallas as pl
from jax.experimental.pallas import tpu as pltpu

H = 64
W = 64
C = 256
HW = H * W
N = HW * 3
CAP = 2048
PRE_K = 2000
POST_K = 1000
ROWB = 2048
NMS_THRESH = 0.7
SCALE_CLAMP = float(np.log(1000.0 / 16.0))
NEG_BIG = -1e30

_TREES = [
    "((((((((0+1)+2)+3)+4)+5)+6)+7)+8)",
    "(((((((0+(1+2))+3)+4)+5)+6)+7)+8)",
    "(((((((0+1)+(2+3))+4)+5)+6)+7)+8)",
    "(((((((0+1)+2)+(3+4))+5)+6)+7)+8)",
    "(((((((0+1)+2)+3)+(4+5))+6)+7)+8)",
    "(((((((0+1)+2)+3)+4)+(5+6))+7)+8)",
    "(((((((0+1)+2)+3)+4)+5)+(6+7))+8)",
    "(((((((0+1)+2)+3)+4)+5)+6)+(7+8))",
    "((((((0+1)+2)+3)+((4+5)+6))+7)+8)",
    "(((((0+((1+2)+3))+4)+5)+(6+7))+8)",
    "(((((0+1)+((2+3)+4))+5)+6)+(7+8))",
    "(((((0+1)+2)+3)+4)+5)",
    "(((3+((4+5)+6))+7)+8)",
]
_CLASS_ROWS = (
    "ccccccccc", "888888888", "888844444", "444444444", "444444444",
    "444444444", "444444444", "444444444", "444077777", "777777777",
    "777777777", "777777777", "777777777", "777777777", "777666666",
    "666666666", "666666666", "666666666", "666666666", "666666666",
    "665555555", "555555555", "555555555", "555555555", "555555555",
    "555555555", "544444444", "444444444", "444444444", "444444444",
    "444444444", "444444444", "000000000", "000000000", "000000000",
    "000000000", "000000000", "000000000", "000000000", "000000000",
    "000033333", "333333333", "333333333", "333333333", "333333333",
    "333333333", "333aaaaaa", "aaaaaaaaa", "aaaaaaa22", "222222222",
    "222222222", "222222222", "229999999", "999999999", "999999111",
    "111111111", "111111111", "111111111", "155555555", "555555555",
    "555550000", "000000000", "000000000", "bbbbbbbbb",
)


def _parse_tree(s, pos=0):
    if s[pos] != "(":
        return int(s[pos]), pos + 1
    left, pos = _parse_tree(s, pos + 1)
    assert s[pos] == "+"
    right, pos = _parse_tree(s, pos + 1)
    assert s[pos] == ")"
    return (left, right), pos + 1


_TREE_AST = [_parse_tree(s)[0] for s in _TREES]
_cm = np.array([[int(c, 16) for c in row] for row in _CLASS_ROWS])
_p = np.arange(HW)
_CLS_NP = _cm[_p // W, (_p % W) % 9].astype(np.int32).reshape(HW, 1)


def _eval_tree(t, parts):
    if isinstance(t, int):
        return parts[t]
    return _eval_tree(t[0], parts) + _eval_tree(t[1], parts)


def _shifted(xf, k):
    kh, kw = k // 3, k % 3
    dy, dx = kh - 1, kw - 1
    off = dy * W + dx
    if off > 0:
        sh = jnp.concatenate([xf[off:], jnp.zeros((off, C), xf.dtype)], axis=0)
    elif off < 0:
        sh = jnp.concatenate([jnp.zeros((-off, C), xf.dtype), xf[:off]], axis=0)
    else:
        sh = xf
    if dx != 0:
        p = lax.broadcasted_iota(jnp.int32, (HW, 1), 0)
        col = p % W + dx
        sh = jnp.where((col >= 0) & (col < W), sh, 0.0)
    return sh


def _dotf(a, b):
    return lax.dot_general(a, b, (((1,), (0,)), ((), ())),
                           preferred_element_type=jnp.float32)


CHUNK = 1024


def conv_head_body(x_ref, w_ref, b1_ref, w2_ref, b2_ref, cls_ref, o_ref):
    g = pl.program_id(0)
    xv = x_ref[pl.ds(g * CHUNK, CHUNK + 256), :]
    prow = lax.broadcasted_iota(jnp.int32, (CHUNK, 1), 0) + g * CHUNK
    parts = []
    for k in range(9):
        kh, kw = k // 3, k % 3
        dy, dx = kh - 1, kw - 1
        off = dy * W + dx
        sh = xv[128 + off:128 + off + CHUNK, :]
        if dx != 0:
            col = prow % W + dx
            sh = jnp.where((col >= 0) & (col < W), sh, 0.0)
        wk = w_ref[k * C:(k + 1) * C, :].astype(jnp.bfloat16)
        parts.append(_dotf(sh.astype(jnp.bfloat16), wk))
    cls = cls_ref[...]
    acc = jnp.zeros((CHUNK, C), jnp.float32)
    for cid, ast in enumerate(_TREE_AST):
        acc = jnp.where(cls == cid, _eval_tree(ast, parts), acc)
    t = jnp.maximum(acc + b1_ref[...], 0.0)
    o_ref[...] = _dotf(t.astype(jnp.bfloat16),
                       w2_ref[...].astype(jnp.bfloat16)) + b2_ref[...]


def decode_body(dl_ref, an_ref, o_ref):
    deltas = dl_ref[...]
    a = an_ref[...]
    widths = a[:, 2] - a[:, 0]
    heights = a[:, 3] - a[:, 1]
    ctr_x = a[:, 0] + 0.5 * widths
    ctr_y = a[:, 1] + 0.5 * heights
    dx, dy = deltas[:, 0], deltas[:, 1]
    dw = jnp.minimum(deltas[:, 2], SCALE_CLAMP)
    dh = jnp.minimum(deltas[:, 3], SCALE_CLAMP)
    pcx = dx * widths + ctr_x
    pcy = dy * heights + ctr_y
    pw = jnp.exp(dw) * widths
    ph = jnp.exp(dh) * heights
    x1 = jnp.clip(pcx - 0.5 * pw, 0.0, 512.0)
    y1 = jnp.clip(pcy - 0.5 * ph, 0.0, 512.0)
    x2 = jnp.clip(pcx + 0.5 * pw, 0.0, 512.0)
    y2 = jnp.clip(pcy + 0.5 * ph, 0.0, 512.0)
    o_ref[...] = jnp.stack([x1, y1, x2, y2], axis=1)


def _monokey(s):
    b = lax.bitcast_convert_type(s, jnp.int32)
    return jnp.where(b < 0, b ^ jnp.int32(0x7FFFFFFF), b)


def rank_all_body(col_ref, row_ref, o_ref):
    g = pl.program_id(0)
    c = pl.program_id(1)
    ka = _monokey(col_ref[...])
    kb = _monokey(row_ref[...])
    ia = lax.broadcasted_iota(jnp.int32, (ROWB, 1), 0) + g * ROWB
    ib = lax.broadcasted_iota(jnp.int32, (1, CAP), 1) + c * CAP
    gt = (kb > ka) | ((kb == ka) & (ib < ia))
    part = jnp.sum(gt.astype(jnp.int32), axis=1, keepdims=True)

    @pl.when(c == 0)
    def _():
        o_ref[...] = part

    @pl.when(c != 0)
    def _():
        o_ref[...] = o_ref[...] + part


def _split3(v):
    a = v.astype(jnp.bfloat16)
    r1 = v - a.astype(jnp.float32)
    b = r1.astype(jnp.bfloat16)
    cc = (r1 - b.astype(jnp.float32)).astype(jnp.bfloat16)
    return a, b, cc


def sel_body(rank_ref, vals_ref, o_ref):
    c = pl.program_id(0)
    rk = rank_ref[...]
    r_iota = lax.broadcasted_iota(jnp.int32, (CAP, 1), 0)
    oh = (rk == r_iota).astype(jnp.bfloat16)
    va, vb, vc = _split3(vals_ref[...])
    part = (_dotf(oh, va) + _dotf(oh, vb)) + _dotf(oh, vc)

    @pl.when(c == 0)
    def _():
        o_ref[...] = part

    @pl.when(c != 0)
    def _():
        o_ref[...] = o_ref[...] + part


def nms_body(bc_ref, sc_ref, o_ref, iou_ref):
    colmask = lax.broadcasted_iota(jnp.int32, (CAP, 1), 0) < PRE_K
    boxes = jnp.where(colmask, bc_ref[...], 0.0)
    area = jnp.maximum(boxes[:, 2] - boxes[:, 0], 0.0) * \
        jnp.maximum(boxes[:, 3] - boxes[:, 1], 0.0)
    RB = 256
    for rb in range(0, CAP, RB):
        bb = boxes[rb:rb + RB]
        ltx = jnp.maximum(bb[:, None, 0], boxes[None, :, 0])
        lty = jnp.maximum(bb[:, None, 1], boxes[None, :, 1])
        rbx = jnp.minimum(bb[:, None, 2], boxes[None, :, 2])
        rby = jnp.minimum(bb[:, None, 3], boxes[None, :, 3])
        wh_x = jnp.maximum(rbx - ltx, 0.0)
        wh_y = jnp.maximum(rby - lty, 0.0)
        inter = wh_x * wh_y
        union = area[rb:rb + RB, None] + area[None, :] - inter
        iou_ref[rb:rb + RB, :] = inter / jnp.maximum(union, 1e-9)

    col_i = lax.broadcasted_iota(jnp.int32, (1, CAP), 1)

    def body(g8, alive):
        grp = iou_ref[pl.ds(g8 * 8, 8), :]
        for j in range(8):
            i = g8 * 8 + j
            row = grp[j:j + 1, :]
            ki = jnp.sum(jnp.where((col_i == i) & (alive > 0), 1, 0)) > 0
            supp = (row > NMS_THRESH) & (col_i > i) & ki
            alive = jnp.where(supp, 0, alive)
        return alive

    alive = lax.fori_loop(0, PRE_K // 8, body, jnp.ones((1, CAP), jnp.int32))
    rowmask = col_i < PRE_K
    o_ref[...] = jnp.where(
        rowmask, jnp.where(alive > 0, sc_ref[...], -1e4), NEG_BIG)


def rank2_body(col_ref, row_ref, o_ref):
    a = col_ref[...]
    b = row_ref[...]
    icol = lax.broadcasted_iota(jnp.int32, (CAP, 1), 0)
    irow = lax.broadcasted_iota(jnp.int32, (1, CAP), 1)
    gt = (b > a) | ((b == a) & (irow < icol))
    o_ref[...] = jnp.sum(gt.astype(jnp.int32), axis=1, keepdims=True)


def fin_body(rank_ref, vals_ref, o_ref):
    rk = rank_ref[...]
    r_iota = lax.broadcasted_iota(jnp.int32, (POST_K + 24, 1), 0)
    oh = (rk == r_iota).astype(jnp.bfloat16)
    va, vb, vc = _split3(vals_ref[...])
    o_ref[...] = (_dotf(oh, va) + _dotf(oh, vb)) + _dotf(oh, vc)


def kernel(features, anchors, w1, b1, w_obj, b_obj, w_del, b_del):
    f32 = jnp.float32
    xb = jnp.transpose(features[0], (1, 2, 0)).reshape(HW, C)
    wt = jnp.transpose(w1, (2, 3, 1, 0)).reshape(9 * C, C)
    w2 = jnp.zeros((C, 128), f32)
    w2 = w2.at[:, 0:3].set(w_obj[:, :, 0, 0].T)
    w2 = w2.at[:, 3:15].set(w_del[:, :, 0, 0].T)
    b2 = jnp.zeros((1, 128), f32)
    b2 = b2.at[0, 0:3].set(b_obj)
    b2 = b2.at[0, 3:15].set(b_del)
    cls = jnp.asarray(_CLS_NP)

    xpad = jnp.concatenate(
        [jnp.zeros((128, C), f32), xb, jnp.zeros((128, C), f32)], axis=0)
    out128 = pl.pallas_call(
        conv_head_body,
        grid=(HW // CHUNK,),
        in_specs=[pl.BlockSpec((HW + 256, C), lambda g: (0, 0)),
                  pl.BlockSpec((9 * C, C), lambda g: (0, 0)),
                  pl.BlockSpec((1, C), lambda g: (0, 0)),
                  pl.BlockSpec((C, 128), lambda g: (0, 0)),
                  pl.BlockSpec((1, 128), lambda g: (0, 0)),
                  pl.BlockSpec((CHUNK, 1), lambda g: (g, 0))],
        out_specs=pl.BlockSpec((CHUNK, 128), lambda g: (g, 0)),
        out_shape=jax.ShapeDtypeStruct((HW, 128), f32),
    )(xpad, wt, b1.reshape(1, C), w2, b2, cls)

    logits = out128[:, 0:3].reshape(N)
    deltas4 = out128[:, 3:15].reshape(N, 4)

    boxes = pl.pallas_call(
        decode_body,
        out_shape=jax.ShapeDtypeStruct((N, 4), f32),
    )(deltas4, anchors)

    rank = pl.pallas_call(
        rank_all_body,
        grid=(N // ROWB, N // CAP),
        in_specs=[pl.BlockSpec((ROWB, 1), lambda g, c: (g, 0)),
                  pl.BlockSpec((1, CAP), lambda g, c: (0, c))],
        out_specs=pl.BlockSpec((ROWB, 1), lambda g, c: (g, 0)),
        out_shape=jax.ShapeDtypeStruct((N, 1), jnp.int32),
    )(logits.reshape(N, 1), logits.reshape(1, N))

    vals = jnp.concatenate(
        [logits.reshape(N, 1), boxes, jnp.zeros((N, 3), f32)], axis=1)
    sel = pl.pallas_call(
        sel_body,
        grid=(N // CAP,),
        in_specs=[pl.BlockSpec((1, CAP), lambda c: (0, c)),
                  pl.BlockSpec((CAP, 8), lambda c: (c, 0))],
        out_specs=pl.BlockSpec((CAP, 8), lambda c: (0, 0)),
        out_shape=jax.ShapeDtypeStruct((CAP, 8), f32),
    )(rank.reshape(1, N), vals)

    ssc = sel[:, 0]
    bc = sel[:, 1:5]
    scm = pl.pallas_call(
        nms_body,
        out_shape=jax.ShapeDtypeStruct((1, CAP), f32),
        scratch_shapes=[pltpu.VMEM((CAP, CAP), f32)],
    )(bc, ssc.reshape(1, CAP))

    rank2 = pl.pallas_call(
        rank2_body,
        out_shape=jax.ShapeDtypeStruct((CAP, 1), jnp.int32),
    )(scm.reshape(CAP, 1), scm)

    vals2 = jnp.concatenate(
        [scm.reshape(CAP, 1), bc, jnp.zeros((CAP, 3), f32)], axis=1)
    fin = pl.pallas_call(
        fin_body,
        out_shape=jax.ShapeDtypeStruct((POST_K + 24, 8), f32),
    )(rank2.reshape(1, CAP), vals2)

    return jnp.concatenate(
        [fin[:POST_K, 1:5], fin[:POST_K, 0:1]], axis=1)
